# baseline (speedup 1.0000x reference)
import functools

import jax
import jax.numpy as jnp
from jax import lax
from jax.experimental import pallas as pl
from jax.experimental.pallas import tpu as pltpu
from jax.experimental.pallas import tpu_sc as plsc

CLS_TOKEN_ID = 2


def _xlane(vec, idx):
    dnums = lax.GatherDimensionNumbers(
        offset_dims=(), collapsed_slice_dims=(0,), start_index_map=(0,))
    return lax.gather(vec, idx[:, None], dnums, slice_sizes=(1,),
                      mode=lax.GatherScatterMode.PROMISE_IN_BOUNDS)

_NC = 2
_NS = 16
_L = 16


@functools.partial(jax.jit, static_argnums=(3, 4))
def _pool(ids_flat, ret16, table, B, S):
    D = table.shape[-1]
    rows_per_core = B // _NC
    workers_per_row = _NS // rows_per_core
    chunk = S // workers_per_row
    n_vec = chunk // _L

    mesh = plsc.VectorSubcoreMesh(core_axis_name="c", subcore_axis_name="s")

    @functools.partial(
        pl.kernel,
        out_type=jax.ShapeDtypeStruct((B, D), jnp.float32),
        mesh=mesh,
        scratch_types=[
            pltpu.VMEM((chunk,), jnp.int32),
            pltpu.VMEM((_L,), jnp.int32),
            pltpu.VMEM_SHARED((_NS * _L,), jnp.int32),
            pltpu.VMEM((_NS * _L,), jnp.int32),
            pltpu.VMEM((_L,), jnp.int32),
            pltpu.VMEM((_L,), jnp.int32),
            pltpu.VMEM((rows_per_core, D), jnp.float32),
            pltpu.SemaphoreType.DMA,
        ],
    )
    def sc_kernel(ids_hbm, ret_hbm, table_hbm, out_hbm,
                  ids_v, run_v, shared, part_v, ret_v, idx_v, rows_v, sem):
        c = lax.axis_index("c")
        s = lax.axis_index("s")
        row_local = s // workers_per_row
        b = c * rows_per_core + row_local
        off = b * S + (s % workers_per_row) * chunk

        pltpu.sync_copy(ids_hbm.at[pl.ds(off, chunk)], ids_v)

        lane = lax.iota(jnp.int32, _L)
        run = jnp.full((_L,), -1, jnp.int32)
        for i in range(n_vec):
            v = ids_v[pl.ds(i * _L, _L)]
            pos = lane + (off + i * _L)
            run = jnp.maximum(run, jnp.where(v == CLS_TOKEN_ID, pos, -1))
        run_v[...] = run
        pltpu.sync_copy(run_v, shared.at[pl.ds(s * _L, _L)])
        plsc.subcore_barrier()

        @pl.when(s == 0)
        def _leader():
            pltpu.sync_copy(shared, part_v)
            pltpu.sync_copy(ret_hbm, ret_v)
            rv = ret_v[...]
            row_idx = []
            for r in range(rows_per_core):
                m = jnp.full((_L,), -1, jnp.int32)
                for j in range(workers_per_row):
                    m = jnp.maximum(
                        m, part_v[pl.ds((r * workers_per_row + j) * _L, _L)])
                m = jnp.where(rv != 0, m, -1)
                for k in (8, 4, 2, 1):
                    m = jnp.maximum(m, _xlane(m, lane ^ k))
                bg = c * rows_per_core + r
                row_idx.append(jnp.where(m < 0, bg * S + (S - 1), m))
            iv = row_idx[-1]
            for r in range(rows_per_core - 1):
                iv = jnp.where(lane == r, row_idx[r], iv)
            idx_v[...] = iv
            pltpu.async_copy(
                table_hbm.at[idx_v.at[pl.ds(0, rows_per_core)]], rows_v, sem
            ).wait()
            pltpu.sync_copy(rows_v, out_hbm.at[pl.ds(c * rows_per_core,
                                                     rows_per_core)])

    return sc_kernel(ids_flat, ret16, table)


def kernel(layer_states, input_ids, return_final):
    B, S = input_ids.shape
    D = layer_states.shape[-1]
    ids_flat = input_ids.reshape(B * S)
    table = layer_states.reshape(B * S, D)
    ret16 = jnp.full((_L,), 1, jnp.int32) * jnp.asarray(return_final, jnp.int32)
    return _pool(ids_flat, ret16, table, B, S)


# speedup vs baseline: 1.0155x; 1.0155x over previous
import functools

import jax
import jax.numpy as jnp
from jax import lax
from jax.experimental import pallas as pl
from jax.experimental.pallas import tpu as pltpu
from jax.experimental.pallas import tpu_sc as plsc

CLS_TOKEN_ID = 2

_NC = 2
_NS = 16
_L = 16


def _xlane(vec, idx):
    dnums = lax.GatherDimensionNumbers(
        offset_dims=(), collapsed_slice_dims=(0,), start_index_map=(0,))
    return lax.gather(vec, idx[:, None], dnums, slice_sizes=(1,),
                      mode=lax.GatherScatterMode.PROMISE_IN_BOUNDS)


@functools.partial(jax.jit, static_argnums=(3,))
def _pool(ids, ret16, table, S):
    B = ids.shape[0]
    D = table.shape[-1]
    rows_per_core = B // _NC
    workers_per_row = _NS // rows_per_core
    chunk = S // workers_per_row
    n_vec = chunk // _L

    mesh = plsc.VectorSubcoreMesh(core_axis_name="c", subcore_axis_name="s")

    @functools.partial(
        pl.kernel,
        out_type=jax.ShapeDtypeStruct((B, D), jnp.float32),
        mesh=mesh,
        scratch_types=[
            pltpu.VMEM((chunk,), jnp.int32),
            pltpu.VMEM((_L,), jnp.int32),
            pltpu.VMEM_SHARED((_NS * _L,), jnp.int32),
            pltpu.VMEM((_NS * _L,), jnp.int32),
            pltpu.VMEM((_L,), jnp.int32),
            pltpu.VMEM((_L,), jnp.int32),
            pltpu.VMEM((rows_per_core, D), jnp.float32),
            pltpu.SemaphoreType.DMA,
        ],
    )
    def sc_kernel(ids_hbm, ret_hbm, table_hbm, out_hbm,
                  ids_v, run_v, shared, part_v, ret_v, idx_v, rows_v, sem):
        c = lax.axis_index("c")
        s = lax.axis_index("s")
        row_local = s // workers_per_row
        b = c * rows_per_core + row_local
        col0 = (s % workers_per_row) * chunk

        pltpu.sync_copy(ids_hbm.at[b, pl.ds(col0, chunk)], ids_v)

        lane = lax.iota(jnp.int32, _L)
        base = b * S + col0

        def scan_step(i, carry):
            run, pos = carry
            v = ids_v[pl.ds(i * _L, _L)]
            return jnp.where(v == CLS_TOKEN_ID, pos, run), pos + _L

        run, _ = lax.fori_loop(
            0, n_vec, scan_step,
            (jnp.full((_L,), -1, jnp.int32), lane + base), unroll=4)
        run_v[...] = run
        pltpu.sync_copy(run_v, shared.at[pl.ds(s * _L, _L)])
        plsc.subcore_barrier()

        @pl.when(s == 0)
        def _leader():
            pltpu.sync_copy(shared, part_v)
            pltpu.sync_copy(ret_hbm, ret_v)
            rv = ret_v[...]

            def comb(j, m):
                return jnp.maximum(m, part_v[pl.ds(j * _L, _L)])

            iv = jnp.full((_L,), 0, jnp.int32)
            for r in range(rows_per_core):
                m = lax.fori_loop(r * workers_per_row,
                                  (r + 1) * workers_per_row, comb,
                                  jnp.full((_L,), -1, jnp.int32))
                m = jnp.where(rv != 0, m, -1)
                for k in (8, 4, 2, 1):
                    m = jnp.maximum(m, _xlane(m, lane ^ k))
                bg = c * rows_per_core + r
                m = jnp.where(m < 0, bg * S + (S - 1), m)
                iv = m if r == 0 else jnp.where(lane < r, iv, m)
            idx_v[...] = iv
            pltpu.async_copy(
                table_hbm.at[idx_v.at[pl.ds(0, rows_per_core)]], rows_v, sem
            ).wait()
            pltpu.sync_copy(rows_v, out_hbm.at[pl.ds(c * rows_per_core,
                                                     rows_per_core)])

    return sc_kernel(ids, ret16, table)


def kernel(layer_states, input_ids, return_final):
    B, S = input_ids.shape
    D = layer_states.shape[-1]
    table = layer_states.reshape(B * S, D)
    ret16 = jnp.broadcast_to(jnp.asarray(return_final, jnp.int32), (_L,))
    return _pool(input_ids, ret16, table, S)


# speedup vs baseline: 1.0284x; 1.0127x over previous
import functools

import jax
import jax.numpy as jnp
from jax import lax
from jax.experimental import pallas as pl
from jax.experimental.pallas import tpu as pltpu
from jax.experimental.pallas import tpu_sc as plsc

CLS_TOKEN_ID = 2

_NC = 2
_NS = 16
_L = 16


def _xlane(vec, idx):
    dnums = lax.GatherDimensionNumbers(
        offset_dims=(), collapsed_slice_dims=(0,), start_index_map=(0,))
    return lax.gather(vec, idx[:, None], dnums, slice_sizes=(1,),
                      mode=lax.GatherScatterMode.PROMISE_IN_BOUNDS)


@functools.partial(jax.jit, static_argnums=(3,))
def _pool(ids, cmp16, table, S):
    B = ids.shape[0]
    D = table.shape[-1]
    rows_per_core = B // _NC
    workers_per_row = _NS // rows_per_core
    chunk = S // workers_per_row
    n_vec = chunk // _L

    mesh = plsc.VectorSubcoreMesh(core_axis_name="c", subcore_axis_name="s")

    @functools.partial(
        pl.kernel,
        out_type=jax.ShapeDtypeStruct((B, D), jnp.float32),
        mesh=mesh,
        scratch_types=[
            pltpu.VMEM((chunk,), jnp.int32),
            pltpu.VMEM((_L,), jnp.int32),
            pltpu.VMEM((_L,), jnp.int32),
            pltpu.VMEM_SHARED((_NS * _L,), jnp.int32),
            pltpu.VMEM((_NS * _L,), jnp.int32),
            pltpu.VMEM((_L,), jnp.int32),
            pltpu.VMEM((rows_per_core, D), jnp.float32),
            pltpu.SemaphoreType.DMA,
            pltpu.SemaphoreType.DMA,
        ],
    )
    def sc_kernel(ids_hbm, cmp_hbm, table_hbm, out_hbm,
                  ids_v, cmp_v, run_v, shared, part_v, idx_v, rows_v,
                  sem_a, sem_b):
        c = lax.axis_index("c")
        s = lax.axis_index("s")
        row_local = s // workers_per_row
        b = c * rows_per_core + row_local
        col0 = (s % workers_per_row) * chunk

        cp_cmp = pltpu.async_copy(cmp_hbm, cmp_v, sem_a)
        cp_ids = pltpu.async_copy(ids_hbm.at[b, pl.ds(col0, chunk)], ids_v,
                                  sem_b)
        cp_cmp.wait()
        cp_ids.wait()
        cv = cmp_v[...]

        lane = lax.iota(jnp.int32, _L)
        base = b * S + col0

        def scan_step(i, carry):
            run, pos = carry
            v = ids_v[pl.ds(i * _L, _L)]
            return jnp.where(v == cv, pos, run), pos + _L

        run, _ = lax.fori_loop(
            0, n_vec, scan_step,
            (jnp.full((_L,), -1, jnp.int32), lane + base), unroll=4)
        run_v[...] = run
        pltpu.sync_copy(run_v, shared.at[pl.ds(s * _L, _L)])
        plsc.subcore_barrier()

        @pl.when(s == 0)
        def _leader():
            pltpu.sync_copy(shared, part_v)

            def comb(j, m):
                return jnp.maximum(m, part_v[pl.ds(j * _L, _L)])

            iv = jnp.full((_L,), 0, jnp.int32)
            for r in range(rows_per_core):
                m = lax.fori_loop(r * workers_per_row,
                                  (r + 1) * workers_per_row, comb,
                                  jnp.full((_L,), -1, jnp.int32))
                for k in (8, 4, 2, 1):
                    m = jnp.maximum(m, _xlane(m, lane ^ k))
                bg = c * rows_per_core + r
                m = jnp.where(m < 0, bg * S + (S - 1), m)
                iv = m if r == 0 else jnp.where(lane < r, iv, m)
            idx_v[...] = iv
            pltpu.async_copy(
                table_hbm.at[idx_v.at[pl.ds(0, rows_per_core)]], rows_v,
                sem_a).wait()
            pltpu.sync_copy(rows_v, out_hbm.at[pl.ds(c * rows_per_core,
                                                     rows_per_core)])

    return sc_kernel(ids, cmp16, table)


def kernel(layer_states, input_ids, return_final):
    B, S = input_ids.shape
    D = layer_states.shape[-1]
    table = layer_states.reshape(B * S, D)
    cmp16 = jnp.broadcast_to(
        jnp.where(return_final, CLS_TOKEN_ID, -1).astype(jnp.int32), (_L,))
    return _pool(input_ids, cmp16, table, S)


# speedup vs baseline: 1.0865x; 1.0565x over previous
import functools

import jax
import jax.numpy as jnp
from jax import lax
from jax.experimental import pallas as pl
from jax.experimental.pallas import tpu as pltpu
from jax.experimental.pallas import tpu_sc as plsc

CLS_TOKEN_ID = 2

_NC = 2
_NS = 16
_L = 16


def _xlane(vec, idx):
    dnums = lax.GatherDimensionNumbers(
        offset_dims=(), collapsed_slice_dims=(0,), start_index_map=(0,))
    return lax.gather(vec, idx[:, None], dnums, slice_sizes=(1,),
                      mode=lax.GatherScatterMode.PROMISE_IN_BOUNDS)


@functools.partial(jax.jit, static_argnums=(3,))
def _pool(ids, cmp16, table, S):
    B = ids.shape[0]
    D = table.shape[-1]
    rows_per_core = B
    workers_per_row = _NS // rows_per_core
    chunk = S // workers_per_row
    n_vec = chunk // _L

    mesh = plsc.VectorSubcoreMesh(core_axis_name="c", subcore_axis_name="s",
                                  num_cores=1)

    @functools.partial(
        pl.kernel,
        out_type=jax.ShapeDtypeStruct((B, D), jnp.float32),
        mesh=mesh,
        scratch_types=[
            pltpu.VMEM((chunk,), jnp.int32),
            pltpu.VMEM((_L,), jnp.int32),
            pltpu.VMEM((_L,), jnp.int32),
            pltpu.VMEM_SHARED((_NS * _L,), jnp.int32),
            pltpu.VMEM((_NS * _L,), jnp.int32),
            pltpu.VMEM((_L,), jnp.int32),
            pltpu.VMEM((rows_per_core, D), jnp.float32),
            pltpu.SemaphoreType.DMA,
            pltpu.SemaphoreType.DMA,
        ],
    )
    def sc_kernel(ids_hbm, cmp_hbm, table_hbm, out_hbm,
                  ids_v, cmp_v, run_v, shared, part_v, idx_v, rows_v,
                  sem_a, sem_b):
        c = lax.axis_index("c")
        s = lax.axis_index("s")
        row_local = s // workers_per_row
        b = c * rows_per_core + row_local
        col0 = (s % workers_per_row) * chunk

        cp_cmp = pltpu.async_copy(cmp_hbm, cmp_v, sem_a)
        cp_ids = pltpu.async_copy(ids_hbm.at[b, pl.ds(col0, chunk)], ids_v,
                                  sem_b)
        cp_cmp.wait()
        cp_ids.wait()
        cv = cmp_v[...]

        lane = lax.iota(jnp.int32, _L)
        base = b * S + col0

        def scan_step(i, carry):
            run, pos = carry
            v = ids_v[pl.ds(i * _L, _L)]
            return jnp.where(v == cv, pos, run), pos + _L

        run, _ = lax.fori_loop(
            0, n_vec, scan_step,
            (jnp.full((_L,), -1, jnp.int32), lane + base), unroll=4)
        run_v[...] = run
        pltpu.sync_copy(run_v, shared.at[pl.ds(s * _L, _L)])
        plsc.subcore_barrier()

        @pl.when(s == 0)
        def _leader():
            pltpu.sync_copy(shared, part_v)

            def comb(j, m):
                return jnp.maximum(m, part_v[pl.ds(j * _L, _L)])

            iv = jnp.full((_L,), 0, jnp.int32)
            for r in range(rows_per_core):
                m = lax.fori_loop(r * workers_per_row,
                                  (r + 1) * workers_per_row, comb,
                                  jnp.full((_L,), -1, jnp.int32))
                for k in (8, 4, 2, 1):
                    m = jnp.maximum(m, _xlane(m, lane ^ k))
                bg = c * rows_per_core + r
                m = jnp.where(m < 0, bg * S + (S - 1), m)
                iv = m if r == 0 else jnp.where(lane < r, iv, m)
            idx_v[...] = iv
            pltpu.async_copy(
                table_hbm.at[idx_v.at[pl.ds(0, rows_per_core)]], rows_v,
                sem_a).wait()
            pltpu.sync_copy(rows_v, out_hbm.at[pl.ds(c * rows_per_core,
                                                     rows_per_core)])

    return sc_kernel(ids, cmp16, table)


def kernel(layer_states, input_ids, return_final):
    B, S = input_ids.shape
    D = layer_states.shape[-1]
    table = layer_states.reshape(B * S, D)
    cmp16 = jnp.broadcast_to(
        jnp.where(return_final, CLS_TOKEN_ID, -1).astype(jnp.int32), (_L,))
    return _pool(input_ids, cmp16, table, S)
